# initial kernel scaffold (unmeasured)
import jax
import jax.numpy as jnp
from jax import lax
from jax.experimental import pallas as pl
from jax.experimental.pallas import tpu as pltpu

NZ = 4


def kernel(O, Wo):
    B, S, Hl, D = O.shape
    K = Hl * D
    N = Wo.shape[1]
    S_loc = S // NZ
    cdt = jnp.bfloat16

    P = jnp.matmul(O.reshape(B, S, K).astype(cdt), Wo.astype(cdt))

    def body(p_ref, out_ref, send_buf, recv_buf, local_buf,
             send_sem, recv_sem, local_sem, out_sem, credit_sem):
        my_x = lax.axis_index("x")
        my_y = lax.axis_index("y")
        my_z = lax.axis_index("z")
        left = lax.rem(my_z + NZ - 1, NZ)
        right = lax.rem(my_z + 1, NZ)

        def chunk(c):
            return p_ref.at[:, pl.ds(lax.rem(c, NZ) * S_loc, S_loc), :]

        bar = pltpu.get_barrier_semaphore()
        pl.semaphore_signal(bar, inc=1, device_id=(my_x, my_y, left),
                            device_id_type=pl.DeviceIdType.MESH)
        pl.semaphore_signal(bar, inc=1, device_id=(my_x, my_y, right),
                            device_id_type=pl.DeviceIdType.MESH)
        pl.semaphore_wait(bar, 2)

        pl.semaphore_signal(credit_sem, inc=1, device_id=(my_x, my_y, left),
                            device_id_type=pl.DeviceIdType.MESH)

        cp = pltpu.make_async_copy(chunk(my_z + NZ - 1), send_buf, local_sem)
        cp.start()
        cp.wait()
        pf = pltpu.make_async_copy(chunk(my_z + NZ - 2), local_buf, local_sem)
        pf.start()

        for t in range(NZ - 1):
            if t > 0:
                pltpu.make_async_copy(
                    chunk(my_z + NZ - 1 - t), local_buf, local_sem).wait()
                send_buf[...] = local_buf[...] + recv_buf[...]
                pl.semaphore_signal(credit_sem, inc=1,
                                    device_id=(my_x, my_y, left),
                                    device_id_type=pl.DeviceIdType.MESH)
                pltpu.make_async_copy(
                    chunk(my_z + 2 * NZ - 2 - t), local_buf, local_sem).start()
            pl.semaphore_wait(credit_sem, 1)
            rdma = pltpu.make_async_remote_copy(
                src_ref=send_buf, dst_ref=recv_buf,
                send_sem=send_sem, recv_sem=recv_sem,
                device_id=(my_x, my_y, right),
                device_id_type=pl.DeviceIdType.MESH)
            rdma.start()
            rdma.wait()

        pltpu.make_async_copy(chunk(my_z), local_buf, local_sem).wait()
        send_buf[...] = local_buf[...] + recv_buf[...]
        ocp = pltpu.make_async_copy(send_buf, out_ref, out_sem)
        ocp.start()
        ocp.wait()

    out = pl.pallas_call(
        body,
        out_shape=jax.ShapeDtypeStruct((B, S_loc, N), cdt),
        in_specs=[pl.BlockSpec(memory_space=pl.ANY)],
        out_specs=pl.BlockSpec(memory_space=pl.ANY),
        scratch_shapes=[
            pltpu.MemorySpace.VMEM((B, S_loc, N), cdt),
            pltpu.MemorySpace.VMEM((B, S_loc, N), cdt),
            pltpu.MemorySpace.VMEM((B, S_loc, N), cdt),
            pltpu.SemaphoreType.DMA,
            pltpu.SemaphoreType.DMA,
            pltpu.SemaphoreType.DMA,
            pltpu.SemaphoreType.DMA,
            pltpu.SemaphoreType.REGULAR,
        ],
        compiler_params=pltpu.CompilerParams(collective_id=0),
    )(P)
    return out.astype(jnp.float32)


# baseline (device time: 1589324 ns/iter reference)
import jax
import jax.numpy as jnp
from jax import lax
from jax.experimental import pallas as pl
from jax.experimental.pallas import tpu as pltpu

try:
    for _a in jax.live_arrays():
        jax.block_until_ready(_a)
except Exception:
    pass

NZ = 4


def kernel(O, Wo):
    B, S, Hl, D = O.shape
    K = Hl * D
    N = Wo.shape[1]
    S_loc = S // NZ
    cdt = jnp.bfloat16

    P = jnp.matmul(O.reshape(B, S, K).astype(cdt), Wo.astype(cdt))

    NSUB = 4
    NS = N // NSUB

    def body(p_ref, out_ref, comm, local_buf,
             send_sems, recv_sems, local_sem, out_sem):
        my_x = lax.axis_index("x")
        my_y = lax.axis_index("y")
        my_z = lax.axis_index("z")
        left = lax.rem(my_z + NZ - 1, NZ)
        right = lax.rem(my_z + 1, NZ)

        bar = pltpu.get_barrier_semaphore()
        pl.semaphore_signal(bar, inc=1, device_id=(my_x, my_y, left),
                            device_id_type=pl.DeviceIdType.MESH)
        pl.semaphore_signal(bar, inc=1, device_id=(my_x, my_y, right),
                            device_id_type=pl.DeviceIdType.MESH)
        pl.semaphore_wait(bar, 2)

        for n in range(NSUB):
            col = pl.ds(n * NS, NS)

            def chunk(c):
                return p_ref.at[:, pl.ds(lax.rem(c, NZ) * S_loc, S_loc), col]

            cp = pltpu.make_async_copy(chunk(my_z + NZ - 1), comm.at[0],
                                       local_sem)
            cp.start()
            cp.wait()
            pltpu.make_async_copy(chunk(my_z + NZ - 2), local_buf,
                                  local_sem).start()

            for h in range(NZ - 1):
                send_slot = h % 2
                recv_slot = (h + 1) % 2
                rdma = pltpu.make_async_remote_copy(
                    src_ref=comm.at[send_slot], dst_ref=comm.at[recv_slot],
                    send_sem=send_sems.at[send_slot],
                    recv_sem=recv_sems.at[recv_slot],
                    device_id=(my_x, my_y, right),
                    device_id_type=pl.DeviceIdType.MESH)
                rdma.start()
                pltpu.make_async_copy(chunk(my_z + NZ - 2 - h), local_buf,
                                      local_sem).wait()
                rdma.wait()
                comm[recv_slot] = comm[recv_slot] + local_buf[...]
                if h < NZ - 2:
                    pltpu.make_async_copy(chunk(my_z + NZ - 3 - h), local_buf,
                                          local_sem).start()

            ocp = pltpu.make_async_copy(comm.at[(NZ - 1) % 2],
                                        out_ref.at[:, :, col], out_sem)
            ocp.start()
            ocp.wait()

    out = pl.pallas_call(
        body,
        out_shape=jax.ShapeDtypeStruct((B, S_loc, N), cdt),
        in_specs=[pl.BlockSpec(memory_space=pl.ANY)],
        out_specs=pl.BlockSpec(memory_space=pl.ANY),
        scratch_shapes=[
            pltpu.MemorySpace.VMEM((2, B, S_loc, NS), cdt),
            pltpu.MemorySpace.VMEM((B, S_loc, NS), cdt),
            pltpu.SemaphoreType.DMA((2,)),
            pltpu.SemaphoreType.DMA((2,)),
            pltpu.SemaphoreType.DMA,
            pltpu.SemaphoreType.DMA,
        ],
        compiler_params=pltpu.CompilerParams(collective_id=0),
    )(P)
    return out.astype(jnp.float32)
